# baseline (speedup 1.0000x reference)
import functools

import jax
import jax.numpy as jnp
import numpy as np
from jax import lax
from jax.experimental import pallas as pl
from jax.experimental.pallas import tpu as pltpu
from jax.experimental.pallas import tpu_sc as plsc

B = 1024
N = 100000
K = 200
NC = 2
NS = 16
NW = NC * NS
TOTAL = B * K
E_PER_W = TOTAL // NW
CHUNK = 128
G_PER_W = E_PER_W // CHUNK
FIRE = 10

_ROWOFF = np.repeat(np.arange(E_PER_W // K, dtype=np.int32) * N, K)


def _gather_body(inp_hbm, idx_hbm, rowoff_hbm, out_hbm,
                 idx_v, rowoff_v, flat_v, out_v, sem):
    wid = lax.axis_index("s") * NC + lax.axis_index("c")
    e_base = wid * E_PER_W

    pltpu.sync_copy(idx_hbm.at[pl.ds(e_base, E_PER_W)], idx_v)
    pltpu.sync_copy(rowoff_hbm, rowoff_v)

    base = jnp.full((16,), wid * (E_PER_W // K) * N, jnp.int32)

    def flat_body(c, _):
        o = c * 16
        flat_v[pl.ds(o, 16)] = (
            idx_v[pl.ds(o, 16)] + rowoff_v[pl.ds(o, 16)] + base)
        return 0

    lax.fori_loop(0, E_PER_W // 16, flat_body, 0)

    def gather_group(grp, _):
        descs = []
        for t in range(FIRE):
            o = (grp * FIRE + t) * CHUNK
            descs.append(pltpu.make_async_copy(
                inp_hbm.at[flat_v.at[pl.ds(o, CHUNK)]],
                out_v.at[pl.ds(o, CHUNK)], sem))
        for d in descs:
            d.start()
        for d in descs:
            d.wait()
        return 0

    lax.fori_loop(0, G_PER_W // FIRE, gather_group, 0)

    pltpu.sync_copy(out_v, out_hbm.at[pl.ds(e_base, E_PER_W)])


@jax.jit
def _gather(inp_flat, idx_flat, rowoff):
    mesh = plsc.VectorSubcoreMesh(core_axis_name="c", subcore_axis_name="s")
    k = functools.partial(
        pl.kernel,
        mesh=mesh,
        out_type=jax.ShapeDtypeStruct((TOTAL,), jnp.float32),
        scratch_types=[
            pltpu.VMEM((E_PER_W,), jnp.int32),
            pltpu.VMEM((E_PER_W,), jnp.int32),
            pltpu.VMEM((E_PER_W,), jnp.int32),
            pltpu.VMEM((E_PER_W,), jnp.float32),
            pltpu.SemaphoreType.DMA,
        ],
    )(_gather_body)
    return k(inp_flat, idx_flat, rowoff)


def kernel(input, index):
    inp_flat = input.reshape(-1)
    idx_flat = index.astype(jnp.int32).reshape(-1)
    rowoff = jnp.asarray(_ROWOFF)
    out = _gather(inp_flat, idx_flat, rowoff)
    return out.reshape(B, K)


# speedup vs baseline: 1.4132x; 1.4132x over previous
import functools

import jax
import jax.numpy as jnp
from jax import lax
from jax.experimental import pallas as pl
from jax.experimental.pallas import tpu as pltpu
from jax.experimental.pallas import tpu_sc as plsc

B = 1024
N = 100000
K = 200
NC = 2
NS = 16
NW = NC * NS
TOTAL = B * K
E_PER_W = TOTAL // NW
ROWS_PER_W = B // NW
GROUPS_PER_W = ROWS_PER_W // 8

W = 6400
NFULL = N // W
TAILC = NFULL * W
TAILW = 3968
SIDEC = TAILC + TAILW
SIDEW = N - SIDEC
NCHUNK = -(-K // 16)
LASTO = K - 16


def _extract(buf_v, idx_v, out_v, i_base, cb, w):
    cb_vec = jnp.full((16,), cb, jnp.int32)
    w_vec = jnp.full((16,), w, jnp.int32)
    zero = jnp.full((16,), 0, jnp.int32)

    def row_body(s, _):
        o_row = i_base + s * K
        row_s = jnp.full((16,), s, jnp.int32)
        for c in range(NCHUNK):
            o = o_row + (c * 16 if c < NCHUNK - 1 else LASTO)
            col = idx_v[pl.ds(o, 16)]
            rel = col - cb_vec
            m = (rel >= zero) & (rel < w_vec)
            v = plsc.load_gather(buf_v, [row_s, rel], mask=m)
            out_v[pl.ds(o, 16)] = jnp.where(m, v, out_v[pl.ds(o, 16)])
        return 0

    lax.fori_loop(0, 8, row_body, 0)


def _gather_body(inp_hbm, side_hbm, idx_hbm, out_hbm,
                 idx_v, buf_v, side_v, out_v, sem):
    wid = lax.axis_index("s") * NC + lax.axis_index("c")
    e_base = wid * E_PER_W
    r_base = wid * ROWS_PER_W

    pltpu.sync_copy(idx_hbm.at[pl.ds(e_base, E_PER_W)], idx_v)
    pltpu.sync_copy(side_hbm.at[pl.ds(r_base, ROWS_PER_W)], side_v)

    def group_body(g, _):
        rg = r_base + g * 8
        i_base = g * 8 * K

        def block_body(b, _):
            cb = b * W
            pltpu.sync_copy(
                inp_hbm.at[pl.ds(rg, 8), pl.ds(cb, W)], buf_v)
            _extract(buf_v, idx_v, out_v, i_base, cb, W)
            return 0

        lax.fori_loop(0, NFULL, block_body, 0)

        pltpu.sync_copy(
            inp_hbm.at[pl.ds(rg, 8), pl.ds(TAILC, TAILW)],
            buf_v.at[:, pl.ds(0, TAILW)])
        _extract(buf_v, idx_v, out_v, i_base, TAILC, TAILW)
        return 0

    lax.fori_loop(0, GROUPS_PER_W, group_body, 0)

    sc_vec = jnp.full((16,), SIDEC, jnp.int32)
    sw_vec = jnp.full((16,), SIDEW, jnp.int32)
    zero = jnp.full((16,), 0, jnp.int32)

    def side_body(s, _):
        o_row = s * K
        row_s = jnp.full((16,), s, jnp.int32)
        for c in range(NCHUNK):
            o = o_row + (c * 16 if c < NCHUNK - 1 else LASTO)
            col = idx_v[pl.ds(o, 16)]
            rel = col - sc_vec
            m = (rel >= zero) & (rel < sw_vec)
            v = plsc.load_gather(side_v, [row_s, rel], mask=m)
            out_v[pl.ds(o, 16)] = jnp.where(m, v, out_v[pl.ds(o, 16)])
        return 0

    lax.fori_loop(0, ROWS_PER_W, side_body, 0)

    pltpu.sync_copy(out_v, out_hbm.at[pl.ds(e_base, E_PER_W)])


@jax.jit
def _gather(inp, side, idx_flat):
    mesh = plsc.VectorSubcoreMesh(core_axis_name="c", subcore_axis_name="s")
    k = functools.partial(
        pl.kernel,
        mesh=mesh,
        out_type=jax.ShapeDtypeStruct((TOTAL,), jnp.float32),
        scratch_types=[
            pltpu.VMEM((E_PER_W,), jnp.int32),
            pltpu.VMEM((8, W), jnp.float32),
            pltpu.VMEM((ROWS_PER_W, SIDEW), jnp.float32),
            pltpu.VMEM((E_PER_W,), jnp.float32),
            pltpu.SemaphoreType.DMA,
        ],
        compiler_params=pltpu.CompilerParams(
            disable_bounds_checks=True, needs_layout_passes=False),
    )(_gather_body)
    return k(inp, side, idx_flat)


def kernel(input, index):
    idx_flat = index.astype(jnp.int32).reshape(-1)
    side = lax.slice(input, (0, SIDEC), (B, N))
    out = _gather(input, side, idx_flat)
    return out.reshape(B, K)


# speedup vs baseline: 1.5746x; 1.1142x over previous
import functools

import jax
import jax.numpy as jnp
from jax import lax
from jax.experimental import pallas as pl
from jax.experimental.pallas import tpu as pltpu
from jax.experimental.pallas import tpu_sc as plsc

B = 1024
N = 100000
K = 200
NC = 2
NS = 16
NW = NC * NS
TOTAL = B * K
E_PER_W = TOTAL // NW
ROWS_PER_W = B // NW
GROUPS_PER_W = ROWS_PER_W // 8

W = 13824
NFULL = N // W
TAILC = NFULL * W
TAILW = 3200
SIDEC = TAILC + TAILW
SIDEW = N - SIDEC
TAILX = TAILW + SIDEW
NCHUNK = -(-K // 16)
LASTO = K - 16


def _extract(buf_v, idx_v, out_v, i_base, cb, w):
    cb_vec = jnp.full((16,), cb, jnp.int32)
    w_vec = jnp.full((16,), w, jnp.uint32)

    def row_body(s, _):
        o_row = i_base + s * K
        row_s = jnp.full((16,), s, jnp.int32)
        for c in range(NCHUNK):
            o = o_row + (c * 16 if c < NCHUNK - 1 else LASTO)
            rel = idx_v[pl.ds(o, 16)] - cb_vec
            m = plsc.bitcast(rel, jnp.uint32) < w_vec
            v = plsc.load_gather(buf_v, [row_s, rel], mask=m)
            out_v[pl.ds(o, 16)] = jnp.where(m, v, out_v[pl.ds(o, 16)])
        return 0

    lax.fori_loop(0, 8, row_body, 0)


def _gather_body(inp_hbm, side_hbm, idx_hbm, out_hbm,
                 idx_v, buf_v, side_v, out_v, sem):
    wid = lax.axis_index("s") * NC + lax.axis_index("c")
    e_base = wid * E_PER_W
    r_base = wid * ROWS_PER_W

    pltpu.sync_copy(idx_hbm.at[pl.ds(e_base, E_PER_W)], idx_v)
    pltpu.sync_copy(side_hbm.at[pl.ds(r_base, ROWS_PER_W)], side_v)

    def group_body(g, _):
        rg = r_base + g * 8
        i_base = g * 8 * K

        def block_body(b, _):
            cb = b * W
            pltpu.sync_copy(
                inp_hbm.at[pl.ds(rg, 8), pl.ds(cb, W)], buf_v)
            _extract(buf_v, idx_v, out_v, i_base, cb, W)
            return 0

        lax.fori_loop(0, NFULL, block_body, 0)

        pltpu.sync_copy(
            inp_hbm.at[pl.ds(rg, 8), pl.ds(TAILC, TAILW)],
            buf_v.at[:, pl.ds(0, TAILW)])

        def fill_body(s, _):
            for c2 in range(SIDEW // 16):
                buf_v[s, pl.ds(TAILW + c2 * 16, 16)] = (
                    side_v[g * 8 + s, pl.ds(c2 * 16, 16)])
            return 0

        lax.fori_loop(0, 8, fill_body, 0)
        _extract(buf_v, idx_v, out_v, i_base, TAILC, TAILX)
        return 0

    lax.fori_loop(0, GROUPS_PER_W, group_body, 0)

    pltpu.sync_copy(out_v, out_hbm.at[pl.ds(e_base, E_PER_W)])


@jax.jit
def _gather(inp, side, idx_flat):
    mesh = plsc.VectorSubcoreMesh(core_axis_name="c", subcore_axis_name="s")
    k = functools.partial(
        pl.kernel,
        mesh=mesh,
        out_type=jax.ShapeDtypeStruct((TOTAL,), jnp.float32),
        scratch_types=[
            pltpu.VMEM((E_PER_W,), jnp.int32),
            pltpu.VMEM((8, W), jnp.float32),
            pltpu.VMEM((ROWS_PER_W, SIDEW), jnp.float32),
            pltpu.VMEM((E_PER_W,), jnp.float32),
            pltpu.SemaphoreType.DMA,
        ],
        compiler_params=pltpu.CompilerParams(
            disable_bounds_checks=True, needs_layout_passes=False),
    )(_gather_body)
    return k(inp, side, idx_flat)


def kernel(input, index):
    idx_flat = index.astype(jnp.int32).reshape(-1)
    side = lax.slice(input, (0, SIDEC), (B, N))
    out = _gather(input, side, idx_flat)
    return out.reshape(B, K)


# speedup vs baseline: 1.5773x; 1.0018x over previous
import functools

import jax
import jax.numpy as jnp
from jax import lax
from jax.experimental import pallas as pl
from jax.experimental.pallas import tpu as pltpu
from jax.experimental.pallas import tpu_sc as plsc

B = 1024
N = 100000
K = 200
NC = 2
NS = 16
NW = NC * NS
TOTAL = B * K
E_PER_W = TOTAL // NW
ROWS_PER_W = B // NW
GROUPS_PER_W = ROWS_PER_W // 8

W = 13824
NFULL = N // W
TAILC = NFULL * W
TAILW = 3200
SIDEC = TAILC + TAILW
SIDEW = N - SIDEC
TAILX = TAILW + SIDEW
NCHUNK = -(-K // 16)
LASTO = K - 16


def _extract(buf_v, idx_v, out_v, i_base, cb, w):
    lanes = lax.iota(jnp.int32, 16)
    cb_vec = jnp.full((16,), cb, jnp.int32)
    w_vec = jnp.full((16,), w, jnp.uint32)

    def row_body(s, _):
        o_row = i_base + s * K
        row_s = jnp.full((16,), s, jnp.int32)
        for c in range(NCHUNK):
            o = o_row + (c * 16 if c < NCHUNK - 1 else LASTO)
            rel = idx_v[pl.ds(o, 16)] - cb_vec
            m = plsc.bitcast(rel, jnp.uint32) < w_vec
            v = plsc.load_gather(buf_v, [row_s, rel], mask=m)
            ovec = jnp.full((16,), o, jnp.int32) + lanes
            plsc.store_scatter(out_v, [ovec], v, mask=m)
        return 0

    lax.fori_loop(0, 8, row_body, 0)


def _gather_body(inp_hbm, side_hbm, idx_hbm, out_hbm,
                 idx_v, buf_v, side_v, out_v, sem):
    wid = lax.axis_index("s") * NC + lax.axis_index("c")
    e_base = wid * E_PER_W
    r_base = wid * ROWS_PER_W

    pltpu.sync_copy(idx_hbm.at[pl.ds(e_base, E_PER_W)], idx_v)
    pltpu.sync_copy(side_hbm.at[pl.ds(r_base, ROWS_PER_W)], side_v)

    def group_body(g, _):
        rg = r_base + g * 8
        i_base = g * 8 * K

        def block_body(b, _):
            cb = b * W
            pltpu.sync_copy(
                inp_hbm.at[pl.ds(rg, 8), pl.ds(cb, W)], buf_v)
            _extract(buf_v, idx_v, out_v, i_base, cb, W)
            return 0

        lax.fori_loop(0, NFULL, block_body, 0)

        pltpu.sync_copy(
            inp_hbm.at[pl.ds(rg, 8), pl.ds(TAILC, TAILW)],
            buf_v.at[:, pl.ds(0, TAILW)])

        def fill_body(s, _):
            for c2 in range(SIDEW // 16):
                buf_v[s, pl.ds(TAILW + c2 * 16, 16)] = (
                    side_v[g * 8 + s, pl.ds(c2 * 16, 16)])
            return 0

        lax.fori_loop(0, 8, fill_body, 0)
        _extract(buf_v, idx_v, out_v, i_base, TAILC, TAILX)
        return 0

    lax.fori_loop(0, GROUPS_PER_W, group_body, 0)

    pltpu.sync_copy(out_v, out_hbm.at[pl.ds(e_base, E_PER_W)])


@jax.jit
def _gather(inp, side, idx_flat):
    mesh = plsc.VectorSubcoreMesh(core_axis_name="c", subcore_axis_name="s")
    k = functools.partial(
        pl.kernel,
        mesh=mesh,
        out_type=jax.ShapeDtypeStruct((TOTAL,), jnp.float32),
        scratch_types=[
            pltpu.VMEM((E_PER_W,), jnp.int32),
            pltpu.VMEM((8, W), jnp.float32),
            pltpu.VMEM((ROWS_PER_W, SIDEW), jnp.float32),
            pltpu.VMEM((E_PER_W,), jnp.float32),
            pltpu.SemaphoreType.DMA,
        ],
        compiler_params=pltpu.CompilerParams(
            disable_bounds_checks=True, needs_layout_passes=False),
    )(_gather_body)
    return k(inp, side, idx_flat)


def kernel(input, index):
    idx_flat = index.astype(jnp.int32).reshape(-1)
    side = lax.slice(input, (0, SIDEC), (B, N))
    out = _gather(input, side, idx_flat)
    return out.reshape(B, K)


# speedup vs baseline: 1.5833x; 1.0038x over previous
import functools

import jax
import jax.numpy as jnp
from jax import lax
from jax.experimental import pallas as pl
from jax.experimental.pallas import tpu as pltpu
from jax.experimental.pallas import tpu_sc as plsc

B = 1024
N = 100000
K = 200
NC = 2
NS = 16
NW = NC * NS
ROWS_PER_W = B // NW
GROUPS_PER_W = ROWS_PER_W // 8

W = 13824
NFULL = N // W
TAILC = NFULL * W
TAILW = 3200
SIDEC = TAILC + TAILW
SIDEW = N - SIDEC
TAILX = TAILW + SIDEW
NCHUNK = -(-K // 16)
LASTO = K - 16


def _extract(buf_v, idx_v, out_v, r0, cb, w):
    lanes = lax.iota(jnp.int32, 16)
    cb_vec = jnp.full((16,), cb, jnp.int32)
    w_vec = jnp.full((16,), w, jnp.uint32)

    def row_body(s, _):
        r = r0 + s
        row_s = jnp.full((16,), s, jnp.int32)
        row_r = jnp.full((16,), r, jnp.int32)
        for c in range(NCHUNK):
            o = c * 16 if c < NCHUNK - 1 else LASTO
            rel = idx_v[r, pl.ds(o, 16)] - cb_vec
            m = plsc.bitcast(rel, jnp.uint32) < w_vec
            v = plsc.load_gather(buf_v, [row_s, rel], mask=m)
            ovec = jnp.full((16,), o, jnp.int32) + lanes
            plsc.store_scatter(out_v, [row_r, ovec], v, mask=m)
        return 0

    lax.fori_loop(0, 8, row_body, 0)


def _gather_body(inp_hbm, side_hbm, idx_hbm, out_hbm,
                 idx_v, buf_v, side_v, out_v, sem):
    wid = lax.axis_index("s") * NC + lax.axis_index("c")
    r_base = wid * ROWS_PER_W

    pltpu.sync_copy(idx_hbm.at[pl.ds(r_base, ROWS_PER_W)], idx_v)
    pltpu.sync_copy(side_hbm.at[pl.ds(r_base, ROWS_PER_W)], side_v)

    def group_body(g, _):
        rg = r_base + g * 8
        r0 = g * 8

        def block_body(b, _):
            cb = b * W
            pltpu.sync_copy(
                inp_hbm.at[pl.ds(rg, 8), pl.ds(cb, W)], buf_v)
            _extract(buf_v, idx_v, out_v, r0, cb, W)
            return 0

        lax.fori_loop(0, NFULL, block_body, 0)

        pltpu.sync_copy(
            inp_hbm.at[pl.ds(rg, 8), pl.ds(TAILC, TAILW)],
            buf_v.at[:, pl.ds(0, TAILW)])

        def fill_body(s, _):
            for c2 in range(SIDEW // 16):
                buf_v[s, pl.ds(TAILW + c2 * 16, 16)] = (
                    side_v[r0 + s, pl.ds(c2 * 16, 16)])
            return 0

        lax.fori_loop(0, 8, fill_body, 0)
        _extract(buf_v, idx_v, out_v, r0, TAILC, TAILX)
        return 0

    lax.fori_loop(0, GROUPS_PER_W, group_body, 0)

    pltpu.sync_copy(out_v, out_hbm.at[pl.ds(r_base, ROWS_PER_W)])


@jax.jit
def _gather(inp, side, idx):
    mesh = plsc.VectorSubcoreMesh(core_axis_name="c", subcore_axis_name="s")
    k = functools.partial(
        pl.kernel,
        mesh=mesh,
        out_type=jax.ShapeDtypeStruct((B, K), jnp.float32),
        scratch_types=[
            pltpu.VMEM((ROWS_PER_W, K), jnp.int32),
            pltpu.VMEM((8, W), jnp.float32),
            pltpu.VMEM((ROWS_PER_W, SIDEW), jnp.float32),
            pltpu.VMEM((ROWS_PER_W, K), jnp.float32),
            pltpu.SemaphoreType.DMA,
        ],
        compiler_params=pltpu.CompilerParams(
            disable_bounds_checks=True, needs_layout_passes=False),
    )(_gather_body)
    return k(inp, side, idx)


def kernel(input, index):
    side = lax.slice(input, (0, SIDEC), (B, N))
    return _gather(input, side, index.astype(jnp.int32))


# speedup vs baseline: 1.5873x; 1.0025x over previous
import functools

import jax
import jax.numpy as jnp
from jax import lax
from jax.experimental import pallas as pl
from jax.experimental.pallas import tpu as pltpu
from jax.experimental.pallas import tpu_sc as plsc

B = 1024
N = 100000
K = 200
NC = 2
NS = 16
NW = NC * NS
ROWS_PER_W = B // NW
GROUPS_PER_W = ROWS_PER_W // 8

W = 13824
NFULL = N // W
TAILC = NFULL * W
TAILW = 3200
SIDEC = TAILC + TAILW
SIDEW = N - SIDEC
TAILX = TAILW + SIDEW
NCHUNK = -(-K // 16)
LASTO = K - 16


def _extract(buf_v, idx_v, out_v, r0, cb, w):
    lanes = lax.iota(jnp.int32, 16)
    cb_vec = jnp.full((16,), cb, jnp.int32)
    w_vec = jnp.full((16,), w, jnp.uint32)

    def row_body(s, _):
        r = r0 + s
        row_s = jnp.full((16,), s, jnp.int32)
        row_r = jnp.full((16,), r, jnp.int32)
        for c in range(NCHUNK):
            o = c * 16 if c < NCHUNK - 1 else LASTO
            rel = idx_v[r, pl.ds(o, 16)] - cb_vec
            m = plsc.bitcast(rel, jnp.uint32) < w_vec
            v = plsc.load_gather(buf_v, [row_s, rel], mask=m)
            ovec = jnp.full((16,), o, jnp.int32) + lanes
            plsc.store_scatter(out_v, [row_r, ovec], v, mask=m)
        return 0

    lax.fori_loop(0, 8, row_body, 0)


def _gather_body(inp_hbm, side_hbm, idx_hbm, out_hbm,
                 idx_v, buf_v, side_v, out_v, sem):
    wid = lax.axis_index("s") * NC + lax.axis_index("c")
    r_base = wid * ROWS_PER_W

    pltpu.sync_copy(idx_hbm.at[pl.ds(r_base, ROWS_PER_W)], idx_v)
    pltpu.sync_copy(side_hbm.at[pl.ds(r_base, ROWS_PER_W)], side_v)

    def group_body(g, _):
        rg = r_base + g * 8
        r0 = g * 8

        def block_body(b, _):
            cb = b * W
            pltpu.sync_copy(
                inp_hbm.at[pl.ds(rg, 8), pl.ds(cb, W)], buf_v)
            _extract(buf_v, idx_v, out_v, r0, cb, W)
            return 0

        lax.fori_loop(0, NFULL, block_body, 0)

        pltpu.sync_copy(
            inp_hbm.at[pl.ds(rg, 8), pl.ds(TAILC, TAILW)],
            buf_v.at[:, pl.ds(0, TAILW)])

        def fill_body(s, _):
            for c2 in range(SIDEW // 16):
                buf_v[s, pl.ds(TAILW + c2 * 16, 16)] = (
                    side_v[r0 + s, pl.ds(c2 * 16, 16)])
            return 0

        lax.fori_loop(0, 8, fill_body, 0)
        _extract(buf_v, idx_v, out_v, r0, TAILC, TAILX)
        return 0

    lax.fori_loop(0, GROUPS_PER_W, group_body, 0)

    pltpu.sync_copy(out_v, out_hbm.at[pl.ds(r_base, ROWS_PER_W)])


@jax.jit
def _gather(inp, side, idx):
    mesh = plsc.VectorSubcoreMesh(core_axis_name="c", subcore_axis_name="s")
    k = functools.partial(
        pl.kernel,
        mesh=mesh,
        out_type=jax.ShapeDtypeStruct((B, K), jnp.float32),
        scratch_types=[
            pltpu.VMEM((ROWS_PER_W, K), jnp.int32),
            pltpu.VMEM((8, W), jnp.float32),
            pltpu.VMEM((ROWS_PER_W, SIDEW), jnp.float32),
            pltpu.VMEM((ROWS_PER_W, K), jnp.float32),
            pltpu.SemaphoreType.DMA,
        ],
        compiler_params=pltpu.CompilerParams(
            disable_bounds_checks=True, needs_layout_passes=False,
            skip_device_barrier=True),
    )(_gather_body)
    return k(inp, side, idx)


def kernel(input, index):
    side = lax.slice(input, (0, SIDEC), (B, N))
    return _gather(input, side, index.astype(jnp.int32))
